# Initial kernel scaffold; baseline (speedup 1.0000x reference)
import jax
import jax.numpy as jnp
from jax.experimental import pallas as pl


def kernel(des, tweet, num_prop, cat_prop, new_feature, edge_index, edge_type, W_des, b_des, W_tweet, b_tweet, W_num, b_num, W_cat, b_cat, W_new, b_new, W_in, b_in, W_r1, att_src1, att_dst1, bias1, W_r2, att_src2, att_dst2, bias2, W_out1, b_out1, W_out2, b_out2):
    raise NotImplementedError("write your pallas kernel here")


# speedup vs baseline: 18.8678x; 18.8678x over previous
import functools

import jax
import jax.numpy as jnp
from jax import lax
from jax.experimental import pallas as pl
from jax.experimental.pallas import tpu as pltpu
from jax.experimental.pallas import tpu_sc as plsc

NN = 10000
EE = 320000
NREL = 2

NCORE, NSUB, LANE = 2, 16, 16
CHUNK = 64
EPAD = 327680
CHUNKS_PER_CORE = EPAD // CHUNK // NCORE
CPT = CHUNKS_PER_CORE // NSUB
NACC = 10112
ROWS_PT = NACC // NSUB

RB = 400


def _leaky(x, slope):
    return jnp.where(x >= 0, x, slope * x)


def _dot(a, b):
    return jnp.dot(a, b, preferred_element_type=jnp.float32)


def _emit_tables(i, x, Wr, Asrc, Adst, xr_out, att_out, amaxs_out, amaxd_out):
    zeros96 = jnp.zeros((x.shape[0], 96), jnp.float32)
    for r in range(NREL):
        xr = _dot(x, Wr[r])
        xr_out[r] = xr
        a_s = _dot(xr, Asrc[r])
        a_d = _dot(xr, Adst[r])
        att_out[r] = jnp.concatenate([a_s, a_d, zeros96], axis=1)
        ms = jnp.broadcast_to(jnp.max(a_s, axis=0, keepdims=True), (8, 16))
        md = jnp.broadcast_to(jnp.max(a_d, axis=0, keepdims=True), (8, 16))

        @pl.when(i == 0)
        def _init(r=r, ms=ms, md=md):
            amaxs_out[r] = ms
            amaxd_out[r] = md

        @pl.when(i > 0)
        def _acc(r=r, ms=ms, md=md):
            amaxs_out[r] = jnp.maximum(amaxs_out[r], ms)
            amaxd_out[r] = jnp.maximum(amaxd_out[r], md)


def _prep_body(des, tweet, nump, catp, newf,
               Wd, bd, Wt, bt, Wn, bn, Wc, bc, Ww, bw,
               Wi0, Wi1, Wi2, Wi3, Wi4, b_in, Wr, Asrc, Adst,
               xr_out, att_out, amaxs_out, amaxd_out):
    i = pl.program_id(0)
    d = _leaky(_dot(des[...], Wd[...]) + bd[...], 0.01)
    t = _leaky(_dot(tweet[...], Wt[...]) + bt[...], 0.01)
    n = _leaky(_dot(nump[...], Wn[...]) + bn[...], 0.01)
    c = _leaky(_dot(catp[...], Wc[...]) + bc[...], 0.01)
    w = _leaky(_dot(newf[...], Ww[...]) + bw[...], 0.01)
    x = _leaky(_dot(d, Wi0[...]) + _dot(t, Wi1[...]) + _dot(n, Wi2[...])
               + _dot(c, Wi3[...]) + _dot(w, Wi4[...]) + b_in[...], 0.01)
    _emit_tables(i, x, Wr, Asrc, Adst, xr_out, att_out, amaxs_out, amaxd_out)


def _prep_call(des, tweet, nump, catp, newf, Wd, bd, Wt, bt, Wn, bn, Wc, bc,
               Ww, bw, Wi0, Wi1, Wi2, Wi3, Wi4, b_in, Wr, Asrc, Adst):
    grid = (NN // RB,)
    row = lambda i: (i, 0)
    def whole(shape):
        return pl.BlockSpec(shape, lambda i: tuple(0 for _ in shape))
    in_specs = [
        pl.BlockSpec((RB, 768), row), pl.BlockSpec((RB, 768), row),
        pl.BlockSpec((RB, 7), row), pl.BlockSpec((RB, 11), row),
        pl.BlockSpec((RB, 1), row),
        whole((768, 28)), whole((1, 28)), whole((768, 36)), whole((1, 36)),
        whole((7, 12)), whole((1, 12)), whole((11, 40)), whole((1, 40)),
        whole((1, 12)), whole((1, 12)),
        whole((28, 128)), whole((36, 128)), whole((12, 128)),
        whole((40, 128)), whole((12, 128)), whole((1, 128)),
        whole((NREL, 128, 128)), whole((NREL, 128, 16)), whole((NREL, 128, 16)),
    ]
    out_specs = [
        pl.BlockSpec((NREL, RB, 128), lambda i: (0, i, 0)),
        pl.BlockSpec((NREL, RB, 128), lambda i: (0, i, 0)),
        pl.BlockSpec((NREL, 8, 16), lambda i: (0, 0, 0)),
        pl.BlockSpec((NREL, 8, 16), lambda i: (0, 0, 0)),
    ]
    out_shape = [
        jax.ShapeDtypeStruct((NREL, NN, 128), jnp.float32),
        jax.ShapeDtypeStruct((NREL, NN, 128), jnp.float32),
        jax.ShapeDtypeStruct((NREL, 8, 16), jnp.float32),
        jax.ShapeDtypeStruct((NREL, 8, 16), jnp.float32),
    ]
    return pl.pallas_call(_prep_body, grid=grid, in_specs=in_specs,
                          out_specs=out_specs, out_shape=out_shape)(
        des, tweet, nump, catp, newf, Wd, bd, Wt, bt, Wn, bn, Wc, bc, Ww, bw,
        Wi0, Wi1, Wi2, Wi3, Wi4, b_in, Wr, Asrc, Adst)


def _make_edge_kernel(heads, out_ch):
    groups = out_ch // LANE

    mesh = plsc.VectorSubcoreMesh(core_axis_name="c", subcore_axis_name="s")

    @functools.partial(
        pl.kernel,
        mesh=mesh,
        out_type=[jax.ShapeDtypeStruct((NCORE, NACC, 128), jnp.float32),
                  jax.ShapeDtypeStruct((EPAD, 16), jnp.float32)],
        scratch_types=[
            pltpu.VMEM((CHUNK,), jnp.int32),
            pltpu.VMEM((CHUNK,), jnp.int32),
            pltpu.VMEM((CHUNK,), jnp.int32),
            pltpu.VMEM((CHUNK,), jnp.int32),
            pltpu.VMEM((CHUNK,), jnp.int32),
            pltpu.VMEM((16,), jnp.float32),
            pltpu.VMEM((CHUNK, 128), jnp.float32),
            pltpu.VMEM((CHUNK, 128), jnp.float32),
            pltpu.VMEM((CHUNK, 16), jnp.float32),
            pltpu.VMEM((CHUNK, 128), jnp.float32),
            pltpu.VMEM_SHARED((NACC, 128), jnp.float32),
            pltpu.SemaphoreType.DMA,
            pltpu.SemaphoreType.DMA,
            pltpu.SemaphoreType.DMA,
        ],
    )
    def edge_kernel(srcp, dstp, etp, xr, att, amaxs, amaxd, zu,
                    u_out, e_out,
                    vsrc, vdst, vet, idx_st, idx_dt, mbuf, arow_s, arow_d,
                    e_r, xrow, u_acc, sem1, sem2, sem3):
        core = lax.axis_index("c")
        sub = lax.axis_index("s")

        base_row = sub * ROWS_PT

        pltpu.sync_copy(zu, xrow)
        for off_, nr in _acc_blocks():
            pltpu.sync_copy(xrow.at[pl.ds(0, nr)],
                            u_acc.at[pl.ds(base_row + off_, nr)])

        pltpu.sync_copy(amaxs.at[0, 0], mbuf)
        ms0 = mbuf[...]
        pltpu.sync_copy(amaxd.at[0, 0], mbuf)
        md0 = mbuf[...]
        pltpu.sync_copy(amaxs.at[1, 0], mbuf)
        ms1 = mbuf[...]
        pltpu.sync_copy(amaxd.at[1, 0], mbuf)
        md1 = mbuf[...]
        l0 = ms0 + md0
        l1 = ms1 + md1
        cvec = jnp.maximum(jnp.where(l0 >= 0, l0, 0.2 * l0),
                           jnp.where(l1 >= 0, l1, 0.2 * l1))
        plsc.subcore_barrier()

        base_chunk = core * CHUNKS_PER_CORE + sub * CPT

        def chunk_body(k, carry):
            off = (base_chunk + k) * CHUNK
            pltpu.sync_copy(srcp.at[pl.ds(off, CHUNK)], vsrc)
            pltpu.sync_copy(dstp.at[pl.ds(off, CHUNK)], vdst)
            pltpu.sync_copy(etp.at[pl.ds(off, CHUNK)], vet)
            for g in range(CHUNK // LANE):
                sl = pl.ds(g * LANE, LANE)
                rel = vet[sl] * NN
                idx_st[sl] = vsrc[sl] + rel
                idx_dt[sl] = vdst[sl] + rel
            cp1 = pltpu.async_copy(att.at[idx_st], arow_s, sem1)
            cp2 = pltpu.async_copy(att.at[idx_dt], arow_d, sem2)
            cp3 = pltpu.async_copy(xr.at[idx_st], xrow, sem3)
            cp1.wait()
            cp2.wait()
            cp3.wait()

            def edge_body(i, carry2):
                av = arow_s[i, pl.ds(0, LANE)]
                bv = arow_d[i, pl.ds(LANE, LANE)]
                a = av + bv
                lk = jnp.where(a >= 0, a, 0.2 * a)
                ev = jnp.exp(lk - cvec)
                e_r[i] = ev
                for h in range(heads):
                    sc = ev[h]
                    for gg in range(groups):
                        csl = pl.ds(h * out_ch + gg * LANE, LANE)
                        xrow[i, csl] = xrow[i, csl] * sc
                return carry2

            lax.fori_loop(0, CHUNK, edge_body, 0)
            pltpu.sync_copy(e_r, e_out.at[pl.ds(off, CHUNK)])
            pltpu.sync_copy(xrow, u_acc.at[vdst], add=True)
            return carry

        lax.fori_loop(0, CPT, chunk_body, 0)
        plsc.subcore_barrier()
        for off_, nr in _acc_blocks():
            pltpu.sync_copy(u_acc.at[pl.ds(base_row + off_, nr)],
                            xrow.at[pl.ds(0, nr)])
            pltpu.sync_copy(xrow.at[pl.ds(0, nr)],
                            u_out.at[core, pl.ds(base_row + off_, nr)])

    return edge_kernel


def _acc_blocks():
    nfull = ROWS_PT // CHUNK
    blocks = [(j * CHUNK, CHUNK) for j in range(nfull)]
    tail = ROWS_PT - nfull * CHUNK
    if tail:
        blocks.append((nfull * CHUNK, tail))
    return blocks


def _make_sum_kernel():
    mesh = plsc.VectorSubcoreMesh(core_axis_name="c", subcore_axis_name="s")

    @functools.partial(
        pl.kernel,
        mesh=mesh,
        out_type=jax.ShapeDtypeStruct((NCORE, NACC, 128), jnp.float32),
        scratch_types=[
            pltpu.VMEM((CHUNK,), jnp.int32),
            pltpu.VMEM((CHUNK, 16), jnp.float32),
            pltpu.VMEM((CHUNK, 128), jnp.float32),
            pltpu.VMEM_SHARED((NACC, 128), jnp.float32),
        ],
    )
    def sum_kernel(dstp, e_buf, zu, s_out, vdst, e_r, e128, s_acc):
        core = lax.axis_index("c")
        sub = lax.axis_index("s")
        base_row = sub * ROWS_PT

        pltpu.sync_copy(zu, e128)
        for off_, nr in _acc_blocks():
            pltpu.sync_copy(e128.at[pl.ds(0, nr)],
                            s_acc.at[pl.ds(base_row + off_, nr)])
        plsc.subcore_barrier()

        base_chunk = core * CHUNKS_PER_CORE + sub * CPT

        def chunk_body(k, carry):
            off = (base_chunk + k) * CHUNK
            pltpu.sync_copy(dstp.at[pl.ds(off, CHUNK)], vdst)
            pltpu.sync_copy(e_buf.at[pl.ds(off, CHUNK)], e_r)

            def cl(i, carry2):
                e128[i, pl.ds(0, 16)] = e_r[i]
                return carry2

            lax.fori_loop(0, CHUNK, cl, 0)
            pltpu.sync_copy(e128, s_acc.at[vdst], add=True)
            return carry

        lax.fori_loop(0, CPT, chunk_body, 0)
        plsc.subcore_barrier()
        for off_, nr in _acc_blocks():
            pltpu.sync_copy(s_acc.at[pl.ds(base_row + off_, nr)],
                            e128.at[pl.ds(0, nr)])
            pltpu.sync_copy(e128.at[pl.ds(0, nr)],
                            s_out.at[core, pl.ds(base_row + off_, nr)])

    return sum_kernel


def _mid_body(u, s, sel, bias, Wr, Asrc, Adst,
              xr_out, att_out, amaxs_out, amaxd_out):
    i = pl.program_id(0)
    u_sum = u[0] + u[1]
    s_sum = s[0] + s[1]
    s128 = _dot(s_sum, sel[...])
    out1 = u_sum / (s128 + 1e-16) + bias[...]
    _emit_tables(i, out1, Wr, Asrc, Adst, xr_out, att_out, amaxs_out, amaxd_out)


def _mid_call(u, s, sel, bias, Wr, Asrc, Adst):
    grid = (NN // RB,)
    def whole(shape):
        return pl.BlockSpec(shape, lambda i: tuple(0 for _ in shape))
    in_specs = [
        pl.BlockSpec((NCORE, RB, 128), lambda i: (0, i, 0)),
        pl.BlockSpec((NCORE, RB, 16), lambda i: (0, i, 0)),
        whole((16, 128)), whole((1, 128)),
        whole((NREL, 128, 128)), whole((NREL, 128, 16)), whole((NREL, 128, 16)),
    ]
    out_specs = [
        pl.BlockSpec((NREL, RB, 128), lambda i: (0, i, 0)),
        pl.BlockSpec((NREL, RB, 128), lambda i: (0, i, 0)),
        pl.BlockSpec((NREL, 8, 16), lambda i: (0, 0, 0)),
        pl.BlockSpec((NREL, 8, 16), lambda i: (0, 0, 0)),
    ]
    out_shape = [
        jax.ShapeDtypeStruct((NREL, NN, 128), jnp.float32),
        jax.ShapeDtypeStruct((NREL, NN, 128), jnp.float32),
        jax.ShapeDtypeStruct((NREL, 8, 16), jnp.float32),
        jax.ShapeDtypeStruct((NREL, 8, 16), jnp.float32),
    ]
    return pl.pallas_call(_mid_body, grid=grid, in_specs=in_specs,
                          out_specs=out_specs, out_shape=out_shape)(
        u, s, sel, bias, Wr, Asrc, Adst)


def _final_body(u, s, sel, bias, W1, b1, W2, b2, out):
    u_sum = u[0] + u[1]
    s_sum = s[0] + s[1]
    s128 = _dot(s_sum, sel[...])
    x = u_sum / (s128 + 1e-16) + bias[...]
    y = _leaky(_dot(x, W1[...]) + b1[...], 0.01)
    out[...] = _dot(y, W2[...]) + b2[...]


def _final_call(u, s, sel, bias, W1, b1, W2, b2):
    grid = (NN // RB,)
    def whole(shape):
        return pl.BlockSpec(shape, lambda i: tuple(0 for _ in shape))
    in_specs = [
        pl.BlockSpec((NCORE, RB, 128), lambda i: (0, i, 0)),
        pl.BlockSpec((NCORE, RB, 16), lambda i: (0, i, 0)),
        whole((16, 128)), whole((1, 128)),
        whole((128, 128)), whole((1, 128)), whole((128, 128)), whole((1, 128)),
    ]
    return pl.pallas_call(
        _final_body, grid=grid, in_specs=in_specs,
        out_specs=pl.BlockSpec((RB, 128), lambda i: (i, 0)),
        out_shape=jax.ShapeDtypeStruct((NN, 128), jnp.float32))(
        u, s, sel, bias, W1, b1, W2, b2)


def kernel(des, tweet, num_prop, cat_prop, new_feature, edge_index, edge_type,
           W_des, b_des, W_tweet, b_tweet, W_num, b_num, W_cat, b_cat,
           W_new, b_new, W_in, b_in, W_r1, att_src1, att_dst1, bias1,
           W_r2, att_src2, att_dst2, bias2, W_out1, b_out1, W_out2, b_out2):
    f32 = jnp.float32
    row = lambda b: b.reshape(1, -1).astype(f32)

    Wi0 = W_in[0:28]
    Wi1 = W_in[28:64]
    Wi2 = W_in[64:76]
    Wi3 = W_in[76:116]
    Wi4 = W_in[116:128]

    eye1 = jnp.kron(jnp.eye(4, dtype=f32), jnp.ones((32, 1), f32))
    eye1 = jnp.pad(eye1, ((0, 0), (0, 12)))
    v_s1 = att_src1.reshape(NREL, 128)
    v_d1 = att_dst1.reshape(NREL, 128)
    A1s = v_s1[:, :, None] * eye1[None]
    A1d = v_d1[:, :, None] * eye1[None]
    eye2 = jnp.pad(jnp.ones((128, 1), f32), ((0, 0), (0, 15)))
    v_s2 = att_src2.reshape(NREL, 128)
    v_d2 = att_dst2.reshape(NREL, 128)
    A2s = v_s2[:, :, None] * eye2[None]
    A2d = v_d2[:, :, None] * eye2[None]

    sel1 = jnp.pad(jnp.kron(jnp.eye(4, dtype=f32), jnp.ones((1, 32), f32)),
                   ((0, 12), (0, 0)))
    sel2 = jnp.pad(jnp.ones((1, 128), f32), ((0, 15), (0, 0)))

    pad = EPAD - EE
    srcp = jnp.concatenate([edge_index[0], jnp.zeros((pad,), jnp.int32)])
    dstp = jnp.concatenate([edge_index[1], jnp.full((pad,), NN, jnp.int32)])
    etp = jnp.concatenate([edge_type, jnp.zeros((pad,), jnp.int32)])
    zu = jnp.zeros((CHUNK, 128), f32)

    xr1, att1, amaxs1, amaxd1 = _prep_call(
        des, tweet, num_prop, cat_prop, new_feature,
        W_des, row(b_des), W_tweet, row(b_tweet), W_num, row(b_num),
        W_cat, row(b_cat), W_new, row(b_new),
        Wi0, Wi1, Wi2, Wi3, Wi4, row(b_in), W_r1, A1s, A1d)

    u1, e1 = _make_edge_kernel(4, 32)(
        srcp, dstp, etp, xr1.reshape(NREL * NN, 128),
        att1.reshape(NREL * NN, 128), amaxs1, amaxd1, zu)
    s1 = _make_sum_kernel()(dstp, e1, zu)

    xr2, att2, amaxs2, amaxd2 = _mid_call(
        u1[:, :NN], s1[:, :NN, 0:16], sel1, row(bias1), W_r2, A2s, A2d)

    u2, e2 = _make_edge_kernel(1, 128)(
        srcp, dstp, etp, xr2.reshape(NREL * NN, 128),
        att2.reshape(NREL * NN, 128), amaxs2, amaxd2, zu)
    s2 = _make_sum_kernel()(dstp, e2, zu)

    W2p = jnp.pad(W_out2, ((0, 0), (0, 126)))
    b2p = jnp.pad(b_out2, (0, 126)).reshape(1, 128)
    out = _final_call(u2[:, :NN], s2[:, :NN, 0:16], sel2, row(bias2),
                      W_out1, row(b_out1), W2p, b2p)
    return out[:, :2]
